# baseline (speedup 1.0000x reference)
import functools

import jax
import jax.numpy as jnp
from jax import lax
from jax.experimental import pallas as pl
from jax.experimental.pallas import tpu as pltpu
from jax.experimental.pallas import tpu_sc as plsc

_N = 10000
_E = 320000
_EW = 80
_NCH = _E // _EW
_NT = 16
_NC = 2
_RT = 624
_TAIL0 = _N - _NT * _RT
_HALF = _NCH // 2
_PT_ALL = _NCH // _NT
_PT_HALF = _HALF // _NT


@functools.cache
def _mesh():
  return plsc.VectorSubcoreMesh(core_axis_name="c", subcore_axis_name="s",
                                num_cores=_NC, num_subcores=_NT)


def _fill_rows(ref, nrows, value):
  v16 = jnp.full((16,), value, jnp.float32)

  def row(j, _):
    for k in range(8):
      ref[j, pl.ds(k * 16, 16)] = v16
    return 0

  lax.fori_loop(0, nrows, row, 0)


def _loop(n, body_fn):
  lax.fori_loop(0, n, body_fn, 0)
  return None


def _zero_acc(buf, acc_s, base, sid):
  _fill_rows(buf, 24, 0.0)

  def zrow(i, _):
    pltpu.sync_copy(buf, acc_s.at[pl.ds(base + i * 24, 24)])
    return 0

  lax.fori_loop(0, _RT // 24, zrow, 0)
  pl.when(sid == 0)(lambda: pltpu.sync_copy(
      buf.at[pl.ds(0, _TAIL0)], acc_s.at[pl.ds(_NT * _RT, _TAIL0)]))


def _flush_acc(buf, acc_s, out_o, base, cid, sid):
  def flush(core):
    def orow(i, _):
      r0 = base + i * 24
      pltpu.sync_copy(acc_s.at[pl.ds(r0, 24)], buf)
      pltpu.sync_copy(buf, out_o.at[core].at[pl.ds(r0, 24)])
      return 0

    lax.fori_loop(0, _RT // 24, orow, 0)

    def tail():
      r0 = _NT * _RT
      pltpu.sync_copy(acc_s.at[pl.ds(r0, _TAIL0)], buf.at[pl.ds(0, _TAIL0)])
      pltpu.sync_copy(buf.at[pl.ds(0, _TAIL0)], out_o.at[core].at[pl.ds(r0, _TAIL0)])

    pl.when(sid == 0)(tail)
    return None

  pl.when(cid == 0)(lambda: flush(0))
  pl.when(cid == 1)(lambda: flush(1))


def _sc_ones_body(dst_h, out_o, didx, vals, buf, acc_s):
  cid = lax.axis_index("c")
  sid = lax.axis_index("s")
  base = sid * _RT
  wid = cid * _NT + sid

  _zero_acc(buf, acc_s, base, sid)
  _fill_rows(vals, _EW, 1.0)
  plsc.subcore_barrier()

  def inner(i, _):
    pltpu.sync_copy(dst_h.at[pl.ds((wid * _PT_HALF + i) * _EW, _EW)], didx)
    pltpu.sync_copy(vals, acc_s.at[didx], add=True)
    return 0

  lax.fori_loop(0, _PT_HALF, inner, 0)
  plsc.subcore_barrier()
  _flush_acc(buf, acc_s, out_o, base, cid, sid)


def _pipelined_edges(nchunks, gidx_v, oidx_h, och0, oidx2, gtab, vals2, sems,
                     acc_s):
  sems_g, sems_o, sems_s = sems

  z16i = jnp.zeros((16,), jnp.int32)
  for b in (0, 1):
    for k in range(_EW // 16):
      oidx2[b][pl.ds(k * 16, 16)] = z16i
  _fill_rows(vals2[0], _EW, 0.0)
  _fill_rows(vals2[1], _EW, 0.0)
  for b in (0, 1):
    pltpu.async_copy(vals2[b], acc_s.at[oidx2[b]], sems_s[b], add=True)

  def wait_scat(b):
    pltpu.make_async_copy(vals2[b], acc_s.at[oidx2[b]], sems_s[b]).wait()

  def ag(b, c):
    wait_scat(b)
    pltpu.async_copy(oidx_h.at[pl.ds((och0 + c) * _EW, _EW)], oidx2[b],
                     sems_o[b])
    pltpu.async_copy(gtab.at[gidx_v.at[pl.ds(c * _EW, _EW)]],
                     vals2[b], sems_g[b])

  def drain(b, c):
    pltpu.make_async_copy(oidx_h.at[pl.ds((och0 + c) * _EW, _EW)], oidx2[b],
                          sems_o[b]).wait()
    pltpu.make_async_copy(gtab.at[gidx_v.at[pl.ds(c * _EW, _EW)]],
                          vals2[b], sems_g[b]).wait()
    pltpu.async_copy(vals2[b], acc_s.at[oidx2[b]], sems_s[b], add=True)

  ag(0, 0)
  nloop = (nchunks - 1) // 2

  def it(i2, _):
    c0 = 2 * i2
    ag(1, c0 + 1)
    drain(0, c0)
    ag(0, c0 + 2)
    drain(1, c0 + 1)
    return 0

  lax.fori_loop(0, nloop, it, 0)
  if nchunks % 2 == 1:
    drain(0, nchunks - 1)
  else:
    ag(1, nchunks - 1)
    drain(0, nchunks - 2)
    drain(1, nchunks - 1)
  wait_scat(0)
  wait_scat(1)


_EPT = _PT_HALF * _EW


def _sc_agg_split_body(src_h, dst_h, tab, dep_h, out_o,
                       gidx_v, oidx0, oidx1, vals0, vals1, buf,
                       semg0, semg1, semo0, semo1, sems0, sems1, acc_s):
  del dep_h
  cid = lax.axis_index("c")
  sid = lax.axis_index("s")
  base = sid * _RT
  wid = cid * _NT + sid

  _zero_acc(buf, acc_s, base, sid)
  pltpu.sync_copy(src_h.at[pl.ds(wid * _EPT, _EPT)], gidx_v)
  plsc.subcore_barrier()

  _pipelined_edges(_PT_HALF, gidx_v, dst_h, wid * _PT_HALF, (oidx0, oidx1),
                   tab, (vals0, vals1),
                   ((semg0, semg1), (semo0, semo1), (sems0, sems1)), acc_s)
  plsc.subcore_barrier()
  _flush_acc(buf, acc_s, out_o, base, cid, sid)


def _sc_agg_cols_body(src_h, dst_h, tabl, tabr, dep_h, out_o,
                      gidx_v, oidx0, oidx1, vals0, vals1, buf,
                      semg0, semg1, semo0, semo1, sems0, sems1, acc_s):
  del dep_h
  cid = lax.axis_index("c")
  sid = lax.axis_index("s")
  base = sid * _RT

  _zero_acc(buf, acc_s, base, sid)
  pltpu.sync_copy(src_h.at[pl.ds(sid * 2 * _EPT, 2 * _EPT)], gidx_v)
  plsc.subcore_barrier()

  def run(tab):
    _pipelined_edges(_PT_ALL, gidx_v, dst_h, sid * _PT_ALL, (oidx0, oidx1),
                     tab, (vals0, vals1),
                     ((semg0, semg1), (semo0, semo1), (sems0, sems1)), acc_s)
    return None

  pl.when(cid == 0)(lambda: run(tabl))
  pl.when(cid == 1)(lambda: run(tabr))
  plsc.subcore_barrier()
  _flush_acc(buf, acc_s, out_o, base, cid, sid)


def _sc_s_body(src_h, dst_h, invt, dep_h, out_o,
               gidx_v, oidx0, oidx1, vals0, vals1, buf,
               semg0, semg1, semo0, semo1, sems0, sems1, acc_s):
  del dep_h
  cid = lax.axis_index("c")
  sid = lax.axis_index("s")
  base = sid * _RT
  wid = cid * _NT + sid

  _zero_acc(buf, acc_s, base, sid)
  pltpu.sync_copy(dst_h.at[pl.ds(wid * _EPT, _EPT)], gidx_v)
  plsc.subcore_barrier()

  _pipelined_edges(_PT_HALF, gidx_v, src_h, wid * _PT_HALF, (oidx0, oidx1),
                   invt, (vals0, vals1),
                   ((semg0, semg1), (semo0, semo1), (sems0, sems1)), acc_s)
  plsc.subcore_barrier()
  _flush_acc(buf, acc_s, out_o, base, cid, sid)


_SC_OUT = [jax.ShapeDtypeStruct((_NC, _N, 128), jnp.float32)]


def _pipe_scratch(nidx):
  return [
      pltpu.VMEM((nidx * _EW,), jnp.int32),
      pltpu.VMEM((_EW,), jnp.int32),
      pltpu.VMEM((_EW,), jnp.int32),
      pltpu.VMEM((_EW, 128), jnp.float32),
      pltpu.VMEM((_EW, 128), jnp.float32),
      pltpu.VMEM((24, 128), jnp.float32),
      pltpu.SemaphoreType.DMA,
      pltpu.SemaphoreType.DMA,
      pltpu.SemaphoreType.DMA,
      pltpu.SemaphoreType.DMA,
      pltpu.SemaphoreType.DMA,
      pltpu.SemaphoreType.DMA,
  ]


_SC_SCRATCH_IDX1 = [
    pltpu.VMEM((_EW,), jnp.int32),
    pltpu.VMEM((_EW, 128), jnp.float32),
    pltpu.VMEM((24, 128), jnp.float32),
]


@functools.cache
def _sc_kernels():
  shared = [pltpu.VMEM_SHARED((_N, 128), jnp.float32)]
  ones = pl.kernel(_sc_ones_body, out_type=_SC_OUT, mesh=_mesh(),
                   scratch_types=_SC_SCRATCH_IDX1 + shared)
  agg_split = pl.kernel(_sc_agg_split_body, out_type=_SC_OUT, mesh=_mesh(),
                        scratch_types=_pipe_scratch(_PT_HALF) + shared)
  agg_cols = pl.kernel(_sc_agg_cols_body, out_type=_SC_OUT, mesh=_mesh(),
                       scratch_types=_pipe_scratch(_PT_ALL) + shared)
  s_pass = pl.kernel(_sc_s_body, out_type=_SC_OUT, mesh=_mesh(),
                     scratch_types=_pipe_scratch(_PT_HALF) + shared)
  return ones, agg_split, agg_cols, s_pass


_R = 400
_G = _N // _R


def _elu(y):
  return jnp.where(y > 0.0, y, jnp.exp(jnp.minimum(y, 0.0)) - 1.0)


def _layer_norm(y, g, b):
  m = jnp.mean(y, axis=-1, keepdims=True)
  v = jnp.mean((y - m) * (y - m), axis=-1, keepdims=True)
  return (y - m) / jnp.sqrt(v + 1e-5) * g + b


def _t0_body(cnt, inv_o):
  inv_o[...] = 1.0 / (cnt[0] + cnt[1] + 1e-8)


def _t1_body(x, agg, inv128, w1t, b1, g1, be1, h1l_o, h1r_o):
  inv = inv128[:, 0:1]
  a = (agg[0] + agg[1]) * inv
  y = jnp.dot(x[...] + a, w1t[...], preferred_element_type=jnp.float32) + b1[...]
  h = _elu(_layer_norm(y, g1[...], be1[...]))
  h1l_o[...] = h[:, :128]
  h1r_o[...] = h[:, 128:]


def _t2_body(h1l, h1r, agg, inv128, s128, w2t, b2, g2, be2, tp_o):
  inv = inv128[:, 0:1]
  h1 = jnp.concatenate([h1l[...], h1r[...]], axis=1)
  a = jnp.concatenate([agg[0], agg[1]], axis=1) * inv
  y = jnp.dot(h1 + a, w2t[...], preferred_element_type=jnp.float32) + b2[...]
  h2 = _elu(_layer_norm(y, g2[...], be2[...]))
  w = (1.0 + s128[0, :, 0] + s128[1, :, 0]) * (1.0 / _N)
  tp_o[...] = jnp.sum(h2 * w[:, None], axis=0)[None, None, :]


def _t3_body(tp, w3t, b3, wm1at, wm1bt, bm1, wm2t, bm2, out_o):
  t = jnp.sum(tp[...][:, 0, :], axis=0, keepdims=True)
  g = jnp.dot(t, w3t[...], preferred_element_type=jnp.float32) + b3[...]
  z = jnp.maximum(
      jnp.dot(g, wm1at[...], preferred_element_type=jnp.float32)
      + jnp.dot(g, wm1bt[...], preferred_element_type=jnp.float32)
      + bm1[...], 0.0)
  out_o[...] = jnp.dot(z, wm2t[...], preferred_element_type=jnp.float32) + bm2[...]


_tc0 = pl.pallas_call(
    _t0_body,
    grid=(_G,),
    in_specs=[pl.BlockSpec((_NC, _R, 128), lambda i: (0, i, 0))],
    out_specs=pl.BlockSpec((_R, 128), lambda i: (i, 0)),
    out_shape=jax.ShapeDtypeStruct((_N, 128), jnp.float32),
)

_tc1_in_specs = [
    pl.BlockSpec((_R, 128), lambda i: (i, 0)),
    pl.BlockSpec((_NC, _R, 128), lambda i: (0, i, 0)),
    pl.BlockSpec((_R, 128), lambda i: (i, 0)),
    pl.BlockSpec((128, 256), lambda i: (0, 0)),
    pl.BlockSpec((1, 256), lambda i: (0, 0)),
    pl.BlockSpec((1, 256), lambda i: (0, 0)),
    pl.BlockSpec((1, 256), lambda i: (0, 0)),
]
_tc1_out_specs = [
    pl.BlockSpec((_R, 128), lambda i: (i, 0)),
    pl.BlockSpec((_R, 128), lambda i: (i, 0)),
]

_tc1 = pl.pallas_call(
    _t1_body,
    grid=(_G,),
    in_specs=_tc1_in_specs,
    out_specs=_tc1_out_specs,
    out_shape=[jax.ShapeDtypeStruct((_N, 128), jnp.float32)] * 2,
)

_tc2_in_specs = [
    pl.BlockSpec((_R, 128), lambda i: (i, 0)),
    pl.BlockSpec((_R, 128), lambda i: (i, 0)),
    pl.BlockSpec((_NC, _R, 128), lambda i: (0, i, 0)),
    pl.BlockSpec((_R, 128), lambda i: (i, 0)),
    pl.BlockSpec((_NC, _R, 128), lambda i: (0, i, 0)),
    pl.BlockSpec((256, 256), lambda i: (0, 0)),
    pl.BlockSpec((1, 256), lambda i: (0, 0)),
    pl.BlockSpec((1, 256), lambda i: (0, 0)),
    pl.BlockSpec((1, 256), lambda i: (0, 0)),
]

_tc2 = pl.pallas_call(
    _t2_body,
    grid=(_G,),
    in_specs=_tc2_in_specs,
    out_specs=pl.BlockSpec((1, 1, 256), lambda i: (i, 0, 0)),
    out_shape=jax.ShapeDtypeStruct((_G, 1, 256), jnp.float32),
)

_tc3_in_specs = [
    pl.BlockSpec((_G, 1, 256), lambda i: (0, 0, 0)),
    pl.BlockSpec((256, 256), lambda i: (0, 0)),
    pl.BlockSpec((1, 256), lambda i: (0, 0)),
    pl.BlockSpec((256, 256), lambda i: (0, 0)),
    pl.BlockSpec((256, 256), lambda i: (0, 0)),
    pl.BlockSpec((1, 256), lambda i: (0, 0)),
    pl.BlockSpec((256, 256), lambda i: (0, 0)),
    pl.BlockSpec((1, 256), lambda i: (0, 0)),
]

_tc3 = pl.pallas_call(
    _t3_body,
    grid=(1,),
    in_specs=_tc3_in_specs,
    out_specs=pl.BlockSpec((1, 256), lambda i: (0, 0)),
    out_shape=jax.ShapeDtypeStruct((1, 256), jnp.float32),
)


@jax.jit
def kernel(x, edge_index, W1, b1, ln1_g, ln1_b, W2, b2, ln2_g, ln2_b,
           W3, b3, Wm1, bm1, Wm2, bm2):
  src = edge_index[0]
  dst = edge_index[1]
  ones_k, agg_split_k, agg_cols_k, s_k = _sc_kernels()

  (cnt2,) = ones_k(dst)
  inv128 = _tc0(cnt2)
  (agg0,) = agg_split_k(src, dst, x, cnt2)
  (s2,) = s_k(src, dst, inv128, agg0)
  h1l, h1r = _tc1(x, agg0, inv128, W1.T, b1[None], ln1_g[None], ln1_b[None])
  (agg1,) = agg_cols_k(src, dst, h1l, h1r, s2)
  tp = _tc2(h1l, h1r, agg1, inv128, s2, W2.T, b2[None], ln2_g[None],
            ln2_b[None])
  out = _tc3(tp, W3.T, b3[None], Wm1[:, :256].T, Wm1[:, 256:].T, bm1[None],
             Wm2.T, bm2[None])
  return out


# speedup vs baseline: 1.0985x; 1.0985x over previous
import functools

import jax
import jax.numpy as jnp
from jax import lax
from jax.experimental import pallas as pl
from jax.experimental.pallas import tpu as pltpu
from jax.experimental.pallas import tpu_sc as plsc

_N = 10000
_E = 320000
_EW = 80
_NCH = _E // _EW
_NT = 16
_NC = 2
_RT = 624
_TAIL0 = _N - _NT * _RT
_HALF = _NCH // 2
_PT_ALL = _NCH // _NT
_PT_HALF = _HALF // _NT


@functools.cache
def _mesh():
  return plsc.VectorSubcoreMesh(core_axis_name="c", subcore_axis_name="s",
                                num_cores=_NC, num_subcores=_NT)


def _fill_rows(ref, nrows, value):
  v16 = jnp.full((16,), value, jnp.float32)

  def row(j, _):
    for k in range(8):
      ref[j, pl.ds(k * 16, 16)] = v16
    return 0

  lax.fori_loop(0, nrows, row, 0)


def _loop(n, body_fn):
  lax.fori_loop(0, n, body_fn, 0)
  return None


def _zero_acc(buf, acc_s, base, sid):
  _fill_rows(buf, 24, 0.0)

  def zrow(i, _):
    pltpu.sync_copy(buf, acc_s.at[pl.ds(base + i * 24, 24)])
    return 0

  lax.fori_loop(0, _RT // 24, zrow, 0)
  pl.when(sid == 0)(lambda: pltpu.sync_copy(
      buf.at[pl.ds(0, _TAIL0)], acc_s.at[pl.ds(_NT * _RT, _TAIL0)]))


def _flush_acc(buf, acc_s, out_o, base, cid, sid):
  def flush(core):
    def orow(i, _):
      r0 = base + i * 24
      pltpu.sync_copy(acc_s.at[pl.ds(r0, 24)], buf)
      pltpu.sync_copy(buf, out_o.at[core].at[pl.ds(r0, 24)])
      return 0

    lax.fori_loop(0, _RT // 24, orow, 0)

    def tail():
      r0 = _NT * _RT
      pltpu.sync_copy(acc_s.at[pl.ds(r0, _TAIL0)], buf.at[pl.ds(0, _TAIL0)])
      pltpu.sync_copy(buf.at[pl.ds(0, _TAIL0)], out_o.at[core].at[pl.ds(r0, _TAIL0)])

    pl.when(sid == 0)(tail)
    return None

  pl.when(cid == 0)(lambda: flush(0))
  pl.when(cid == 1)(lambda: flush(1))


def _sc_ones_body(dst_h, out_o, didx0, didx1, vals, buf, semo0, semo1, acc_s):
  cid = lax.axis_index("c")
  sid = lax.axis_index("s")
  base = sid * _RT
  wid = cid * _NT + sid

  _zero_acc(buf, acc_s, base, sid)
  _fill_rows(vals, _EW, 1.0)
  plsc.subcore_barrier()

  didx2 = (didx0, didx1)
  sems = (semo0, semo1)

  def ag(b, c):
    pltpu.async_copy(dst_h.at[pl.ds((wid * _PT_HALF + c) * _EW, _EW)],
                     didx2[b], sems[b])

  def drain(b, c):
    pltpu.make_async_copy(dst_h.at[pl.ds((wid * _PT_HALF + c) * _EW, _EW)],
                          didx2[b], sems[b]).wait()
    pltpu.sync_copy(vals, acc_s.at[didx2[b]], add=True)

  ag(0, 0)
  nloop = (_PT_HALF - 1) // 2

  def it(i2, _):
    c0 = 2 * i2
    ag(1, c0 + 1)
    drain(0, c0)
    ag(0, c0 + 2)
    drain(1, c0 + 1)
    return 0

  lax.fori_loop(0, nloop, it, 0)
  drain(0, _PT_HALF - 1)
  plsc.subcore_barrier()
  _flush_acc(buf, acc_s, out_o, base, cid, sid)


def _pipelined_edges(nchunks, gidx_v, oidx_h, och0, oidx2, gtab, vals2, sems,
                     acc_s):
  sems_g, sems_o = sems

  def ag(b, c):
    pltpu.async_copy(oidx_h.at[pl.ds((och0 + c) * _EW, _EW)], oidx2[b],
                     sems_o[b])
    pltpu.async_copy(gtab.at[gidx_v.at[pl.ds(c * _EW, _EW)]],
                     vals2[b], sems_g[b])

  def drain(b, c):
    pltpu.make_async_copy(oidx_h.at[pl.ds((och0 + c) * _EW, _EW)], oidx2[b],
                          sems_o[b]).wait()
    pltpu.make_async_copy(gtab.at[gidx_v.at[pl.ds(c * _EW, _EW)]],
                          vals2[b], sems_g[b]).wait()
    pltpu.sync_copy(vals2[b], acc_s.at[oidx2[b]], add=True)

  ag(0, 0)
  nloop = (nchunks - 1) // 2

  def it(i2, _):
    c0 = 2 * i2
    ag(1, c0 + 1)
    drain(0, c0)
    ag(0, c0 + 2)
    drain(1, c0 + 1)
    return 0

  lax.fori_loop(0, nloop, it, 0)
  if nchunks % 2 == 1:
    drain(0, nchunks - 1)
  else:
    ag(1, nchunks - 1)
    drain(0, nchunks - 2)
    drain(1, nchunks - 1)


_EPT = _PT_HALF * _EW


def _sc_agg_split_body(src_h, dst_h, tab, dep_h, out_o,
                       gidx_v, oidx0, oidx1, vals0, vals1, buf,
                       semg0, semg1, semo0, semo1, acc_s):
  del dep_h
  cid = lax.axis_index("c")
  sid = lax.axis_index("s")
  base = sid * _RT
  wid = cid * _NT + sid

  _zero_acc(buf, acc_s, base, sid)
  pltpu.sync_copy(src_h.at[pl.ds(wid * _EPT, _EPT)], gidx_v)
  plsc.subcore_barrier()

  _pipelined_edges(_PT_HALF, gidx_v, dst_h, wid * _PT_HALF, (oidx0, oidx1),
                   tab, (vals0, vals1),
                   ((semg0, semg1), (semo0, semo1)), acc_s)
  plsc.subcore_barrier()
  _flush_acc(buf, acc_s, out_o, base, cid, sid)


def _sc_agg_cols_body(src_h, dst_h, tabl, tabr, dep_h, out_o,
                      gidx_v, oidx0, oidx1, vals0, vals1, buf,
                      semg0, semg1, semo0, semo1, acc_s):
  del dep_h
  cid = lax.axis_index("c")
  sid = lax.axis_index("s")
  base = sid * _RT

  _zero_acc(buf, acc_s, base, sid)
  pltpu.sync_copy(src_h.at[pl.ds(sid * 2 * _EPT, 2 * _EPT)], gidx_v)
  plsc.subcore_barrier()

  def run(tab):
    _pipelined_edges(_PT_ALL, gidx_v, dst_h, sid * _PT_ALL, (oidx0, oidx1),
                     tab, (vals0, vals1),
                     ((semg0, semg1), (semo0, semo1)), acc_s)
    return None

  pl.when(cid == 0)(lambda: run(tabl))
  pl.when(cid == 1)(lambda: run(tabr))
  plsc.subcore_barrier()
  _flush_acc(buf, acc_s, out_o, base, cid, sid)


def _sc_s_body(src_h, dst_h, invt, dep_h, out_o,
               gidx_v, oidx0, oidx1, vals0, vals1, buf,
               semg0, semg1, semo0, semo1, acc_s):
  del dep_h
  cid = lax.axis_index("c")
  sid = lax.axis_index("s")
  base = sid * _RT
  wid = cid * _NT + sid

  _zero_acc(buf, acc_s, base, sid)
  pltpu.sync_copy(dst_h.at[pl.ds(wid * _EPT, _EPT)], gidx_v)
  plsc.subcore_barrier()

  _pipelined_edges(_PT_HALF, gidx_v, src_h, wid * _PT_HALF, (oidx0, oidx1),
                   invt, (vals0, vals1),
                   ((semg0, semg1), (semo0, semo1)), acc_s)
  plsc.subcore_barrier()
  _flush_acc(buf, acc_s, out_o, base, cid, sid)


_SC_OUT = [jax.ShapeDtypeStruct((_NC, _N, 128), jnp.float32)]


def _pipe_scratch(nidx):
  return [
      pltpu.VMEM((nidx * _EW,), jnp.int32),
      pltpu.VMEM((_EW,), jnp.int32),
      pltpu.VMEM((_EW,), jnp.int32),
      pltpu.VMEM((_EW, 128), jnp.float32),
      pltpu.VMEM((_EW, 128), jnp.float32),
      pltpu.VMEM((24, 128), jnp.float32),
      pltpu.SemaphoreType.DMA,
      pltpu.SemaphoreType.DMA,
      pltpu.SemaphoreType.DMA,
      pltpu.SemaphoreType.DMA,
  ]


_SC_SCRATCH_IDX1 = [
    pltpu.VMEM((_EW,), jnp.int32),
    pltpu.VMEM((_EW,), jnp.int32),
    pltpu.VMEM((_EW, 128), jnp.float32),
    pltpu.VMEM((24, 128), jnp.float32),
    pltpu.SemaphoreType.DMA,
    pltpu.SemaphoreType.DMA,
]


@functools.cache
def _sc_kernels():
  shared = [pltpu.VMEM_SHARED((_N, 128), jnp.float32)]
  ones = pl.kernel(_sc_ones_body, out_type=_SC_OUT, mesh=_mesh(),
                   scratch_types=_SC_SCRATCH_IDX1 + shared)
  agg_split = pl.kernel(_sc_agg_split_body, out_type=_SC_OUT, mesh=_mesh(),
                        scratch_types=_pipe_scratch(_PT_HALF) + shared)
  agg_cols = pl.kernel(_sc_agg_cols_body, out_type=_SC_OUT, mesh=_mesh(),
                       scratch_types=_pipe_scratch(_PT_ALL) + shared)
  s_pass = pl.kernel(_sc_s_body, out_type=_SC_OUT, mesh=_mesh(),
                     scratch_types=_pipe_scratch(_PT_HALF) + shared)
  return ones, agg_split, agg_cols, s_pass


_R = 400
_G = _N // _R


def _elu(y):
  return jnp.where(y > 0.0, y, jnp.exp(jnp.minimum(y, 0.0)) - 1.0)


def _layer_norm(y, g, b):
  m = jnp.mean(y, axis=-1, keepdims=True)
  v = jnp.mean((y - m) * (y - m), axis=-1, keepdims=True)
  return (y - m) / jnp.sqrt(v + 1e-5) * g + b


def _t0_body(cnt, inv_o):
  inv_o[...] = 1.0 / (cnt[0] + cnt[1] + 1e-8)


def _t1_body(x, agg, inv128, w1t, b1, g1, be1, h1l_o, h1r_o):
  inv = inv128[:, 0:1]
  a = (agg[0] + agg[1]) * inv
  y = jnp.dot(x[...] + a, w1t[...], preferred_element_type=jnp.float32) + b1[...]
  h = _elu(_layer_norm(y, g1[...], be1[...]))
  h1l_o[...] = h[:, :128]
  h1r_o[...] = h[:, 128:]


def _t2_body(h1l, h1r, agg, inv128, s128, w2t, b2, g2, be2, tp_o):
  inv = inv128[:, 0:1]
  h1 = jnp.concatenate([h1l[...], h1r[...]], axis=1)
  a = jnp.concatenate([agg[0], agg[1]], axis=1) * inv
  y = jnp.dot(h1 + a, w2t[...], preferred_element_type=jnp.float32) + b2[...]
  h2 = _elu(_layer_norm(y, g2[...], be2[...]))
  w = (1.0 + s128[0, :, 0] + s128[1, :, 0]) * (1.0 / _N)
  tp_o[...] = jnp.sum(h2 * w[:, None], axis=0)[None, None, :]


def _t3_body(tp, w3t, b3, wm1at, wm1bt, bm1, wm2t, bm2, out_o):
  t = jnp.sum(tp[...][:, 0, :], axis=0, keepdims=True)
  g = jnp.dot(t, w3t[...], preferred_element_type=jnp.float32) + b3[...]
  z = jnp.maximum(
      jnp.dot(g, wm1at[...], preferred_element_type=jnp.float32)
      + jnp.dot(g, wm1bt[...], preferred_element_type=jnp.float32)
      + bm1[...], 0.0)
  out_o[...] = jnp.dot(z, wm2t[...], preferred_element_type=jnp.float32) + bm2[...]


_tc0 = pl.pallas_call(
    _t0_body,
    grid=(_G,),
    in_specs=[pl.BlockSpec((_NC, _R, 128), lambda i: (0, i, 0))],
    out_specs=pl.BlockSpec((_R, 128), lambda i: (i, 0)),
    out_shape=jax.ShapeDtypeStruct((_N, 128), jnp.float32),
)

_tc1_in_specs = [
    pl.BlockSpec((_R, 128), lambda i: (i, 0)),
    pl.BlockSpec((_NC, _R, 128), lambda i: (0, i, 0)),
    pl.BlockSpec((_R, 128), lambda i: (i, 0)),
    pl.BlockSpec((128, 256), lambda i: (0, 0)),
    pl.BlockSpec((1, 256), lambda i: (0, 0)),
    pl.BlockSpec((1, 256), lambda i: (0, 0)),
    pl.BlockSpec((1, 256), lambda i: (0, 0)),
]
_tc1_out_specs = [
    pl.BlockSpec((_R, 128), lambda i: (i, 0)),
    pl.BlockSpec((_R, 128), lambda i: (i, 0)),
]

_tc1 = pl.pallas_call(
    _t1_body,
    grid=(_G,),
    in_specs=_tc1_in_specs,
    out_specs=_tc1_out_specs,
    out_shape=[jax.ShapeDtypeStruct((_N, 128), jnp.float32)] * 2,
)

_tc2_in_specs = [
    pl.BlockSpec((_R, 128), lambda i: (i, 0)),
    pl.BlockSpec((_R, 128), lambda i: (i, 0)),
    pl.BlockSpec((_NC, _R, 128), lambda i: (0, i, 0)),
    pl.BlockSpec((_R, 128), lambda i: (i, 0)),
    pl.BlockSpec((_NC, _R, 128), lambda i: (0, i, 0)),
    pl.BlockSpec((256, 256), lambda i: (0, 0)),
    pl.BlockSpec((1, 256), lambda i: (0, 0)),
    pl.BlockSpec((1, 256), lambda i: (0, 0)),
    pl.BlockSpec((1, 256), lambda i: (0, 0)),
]

_tc2 = pl.pallas_call(
    _t2_body,
    grid=(_G,),
    in_specs=_tc2_in_specs,
    out_specs=pl.BlockSpec((1, 1, 256), lambda i: (i, 0, 0)),
    out_shape=jax.ShapeDtypeStruct((_G, 1, 256), jnp.float32),
)

_tc3_in_specs = [
    pl.BlockSpec((_G, 1, 256), lambda i: (0, 0, 0)),
    pl.BlockSpec((256, 256), lambda i: (0, 0)),
    pl.BlockSpec((1, 256), lambda i: (0, 0)),
    pl.BlockSpec((256, 256), lambda i: (0, 0)),
    pl.BlockSpec((256, 256), lambda i: (0, 0)),
    pl.BlockSpec((1, 256), lambda i: (0, 0)),
    pl.BlockSpec((256, 256), lambda i: (0, 0)),
    pl.BlockSpec((1, 256), lambda i: (0, 0)),
]

_tc3 = pl.pallas_call(
    _t3_body,
    grid=(1,),
    in_specs=_tc3_in_specs,
    out_specs=pl.BlockSpec((1, 256), lambda i: (0, 0)),
    out_shape=jax.ShapeDtypeStruct((1, 256), jnp.float32),
)


@jax.jit
def kernel(x, edge_index, W1, b1, ln1_g, ln1_b, W2, b2, ln2_g, ln2_b,
           W3, b3, Wm1, bm1, Wm2, bm2):
  src = edge_index[0]
  dst = edge_index[1]
  ones_k, agg_split_k, agg_cols_k, s_k = _sc_kernels()

  (cnt2,) = ones_k(dst)
  inv128 = _tc0(cnt2)
  (agg0,) = agg_split_k(src, dst, x, cnt2)
  (s2,) = s_k(src, dst, inv128, agg0)
  h1l, h1r = _tc1(x, agg0, inv128, W1.T, b1[None], ln1_g[None], ln1_b[None])
  (agg1,) = agg_cols_k(src, dst, h1l, h1r, s2)
  tp = _tc2(h1l, h1r, agg1, inv128, s2, W2.T, b2[None], ln2_g[None],
            ln2_b[None])
  out = _tc3(tp, W3.T, b3[None], Wm1[:, :256].T, Wm1[:, 256:].T, bm1[None],
             Wm2.T, bm2[None])
  return out


# speedup vs baseline: 1.0998x; 1.0012x over previous
import functools

import jax
import jax.numpy as jnp
from jax import lax
from jax.experimental import pallas as pl
from jax.experimental.pallas import tpu as pltpu
from jax.experimental.pallas import tpu_sc as plsc

_N = 10000
_E = 320000
_EW = 80
_NCH = _E // _EW
_NT = 16
_NC = 2
_RT = 624
_TAIL0 = _N - _NT * _RT
_HALF = _NCH // 2
_PT_ALL = _NCH // _NT
_PT_HALF = _HALF // _NT


@functools.cache
def _mesh():
  return plsc.VectorSubcoreMesh(core_axis_name="c", subcore_axis_name="s",
                                num_cores=_NC, num_subcores=_NT)


def _fill_rows(ref, nrows, value):
  v16 = jnp.full((16,), value, jnp.float32)

  def row(j, _):
    for k in range(8):
      ref[j, pl.ds(k * 16, 16)] = v16
    return 0

  lax.fori_loop(0, nrows, row, 0)


def _zero_acc(buf, acc_s, base, sid):
  _fill_rows(buf, 24, 0.0)

  def zrow(i, _):
    pltpu.sync_copy(buf, acc_s.at[pl.ds(base + i * 24, 24)])
    return 0

  lax.fori_loop(0, _RT // 24, zrow, 0)
  pl.when(sid == 0)(lambda: pltpu.sync_copy(
      buf.at[pl.ds(0, _TAIL0)], acc_s.at[pl.ds(_NT * _RT, _TAIL0)]))


def _flush_acc(buf, acc_s, out_o, base, cid, sid):
  def flush(core):
    def orow(i, _):
      r0 = base + i * 24
      pltpu.sync_copy(acc_s.at[pl.ds(r0, 24)], buf)
      pltpu.sync_copy(buf, out_o.at[core].at[pl.ds(r0, 24)])
      return 0

    lax.fori_loop(0, _RT // 24, orow, 0)

    def tail():
      r0 = _NT * _RT
      pltpu.sync_copy(acc_s.at[pl.ds(r0, _TAIL0)], buf.at[pl.ds(0, _TAIL0)])
      pltpu.sync_copy(buf.at[pl.ds(0, _TAIL0)], out_o.at[core].at[pl.ds(r0, _TAIL0)])

    pl.when(sid == 0)(tail)
    return None

  pl.when(cid == 0)(lambda: flush(0))
  pl.when(cid == 1)(lambda: flush(1))


def _sc_ones_body(dst_h, out_o, didx0, didx1, vals, buf, semo0, semo1, acc_s):
  cid = lax.axis_index("c")
  sid = lax.axis_index("s")
  base = sid * _RT
  wid = cid * _NT + sid

  _zero_acc(buf, acc_s, base, sid)
  _fill_rows(vals, _EW, 1.0)
  plsc.subcore_barrier()

  didx2 = (didx0, didx1)
  sems = (semo0, semo1)

  def ag(b, c):
    pltpu.async_copy(dst_h.at[pl.ds((wid * _PT_HALF + c) * _EW, _EW)],
                     didx2[b], sems[b])

  def drain(b, c):
    pltpu.make_async_copy(dst_h.at[pl.ds((wid * _PT_HALF + c) * _EW, _EW)],
                          didx2[b], sems[b]).wait()
    pltpu.sync_copy(vals, acc_s.at[didx2[b]], add=True)

  ag(0, 0)
  nloop = (_PT_HALF - 1) // 2

  def it(i2, _):
    c0 = 2 * i2
    ag(1, c0 + 1)
    drain(0, c0)
    ag(0, c0 + 2)
    drain(1, c0 + 1)
    return 0

  lax.fori_loop(0, nloop, it, 0)
  drain(0, _PT_HALF - 1)
  plsc.subcore_barrier()
  _flush_acc(buf, acc_s, out_o, base, cid, sid)


def _pipelined_edges(nchunks, gidx_v, oidx_h, och0, oidx2, gtab, vals2, sems,
                     acc_s):
  sems_g, sems_o = sems

  def ag(b, c):
    pltpu.async_copy(oidx_h.at[pl.ds((och0 + c) * _EW, _EW)], oidx2[b],
                     sems_o[b])
    pltpu.async_copy(gtab.at[gidx_v.at[pl.ds(c * _EW, _EW)]],
                     vals2[b], sems_g[b])

  def drain(b, c):
    pltpu.make_async_copy(oidx_h.at[pl.ds((och0 + c) * _EW, _EW)], oidx2[b],
                          sems_o[b]).wait()
    pltpu.make_async_copy(gtab.at[gidx_v.at[pl.ds(c * _EW, _EW)]],
                          vals2[b], sems_g[b]).wait()
    pltpu.sync_copy(vals2[b], acc_s.at[oidx2[b]], add=True)

  ag(0, 0)
  nloop = (nchunks - 1) // 2

  def it(i2, _):
    c0 = 2 * i2
    ag(1, c0 + 1)
    drain(0, c0)
    ag(0, c0 + 2)
    drain(1, c0 + 1)
    return 0

  lax.fori_loop(0, nloop, it, 0)
  if nchunks % 2 == 1:
    drain(0, nchunks - 1)
  else:
    ag(1, nchunks - 1)
    drain(0, nchunks - 2)
    drain(1, nchunks - 1)


_EPT = _PT_HALF * _EW


def _sc_agg_split_body(src_h, dst_h, tab, dep_h, out_o,
                       gidx_v, oidx0, oidx1, vals0, vals1, buf,
                       semg0, semg1, semo0, semo1, acc_s):
  del dep_h
  cid = lax.axis_index("c")
  sid = lax.axis_index("s")
  base = sid * _RT
  wid = cid * _NT + sid

  _zero_acc(buf, acc_s, base, sid)
  pltpu.sync_copy(src_h.at[pl.ds(wid * _EPT, _EPT)], gidx_v)
  plsc.subcore_barrier()

  _pipelined_edges(_PT_HALF, gidx_v, dst_h, wid * _PT_HALF, (oidx0, oidx1),
                   tab, (vals0, vals1),
                   ((semg0, semg1), (semo0, semo1)), acc_s)
  plsc.subcore_barrier()
  _flush_acc(buf, acc_s, out_o, base, cid, sid)


def _sc_agg_cols_body(src_h, dst_h, tabl, tabr, dep_h, out_o,
                      gidx_v, oidx0, oidx1, vals0, vals1, buf,
                      semg0, semg1, semo0, semo1, acc_s):
  del dep_h
  cid = lax.axis_index("c")
  sid = lax.axis_index("s")
  base = sid * _RT

  _zero_acc(buf, acc_s, base, sid)
  pltpu.sync_copy(src_h.at[pl.ds(sid * 2 * _EPT, 2 * _EPT)], gidx_v)
  plsc.subcore_barrier()

  def run(tab):
    _pipelined_edges(_PT_ALL, gidx_v, dst_h, sid * _PT_ALL, (oidx0, oidx1),
                     tab, (vals0, vals1),
                     ((semg0, semg1), (semo0, semo1)), acc_s)
    return None

  pl.when(cid == 0)(lambda: run(tabl))
  pl.when(cid == 1)(lambda: run(tabr))
  plsc.subcore_barrier()
  _flush_acc(buf, acc_s, out_o, base, cid, sid)


def _sc_s_body(src_h, dst_h, invt, dep_h, out_o,
               gidx_v, oidx0, oidx1, vals0, vals1, buf,
               semg0, semg1, semo0, semo1, acc_s):
  del dep_h
  cid = lax.axis_index("c")
  sid = lax.axis_index("s")
  base = sid * _RT
  wid = cid * _NT + sid

  _zero_acc(buf, acc_s, base, sid)
  pltpu.sync_copy(dst_h.at[pl.ds(wid * _EPT, _EPT)], gidx_v)
  plsc.subcore_barrier()

  _pipelined_edges(_PT_HALF, gidx_v, src_h, wid * _PT_HALF, (oidx0, oidx1),
                   invt, (vals0, vals1),
                   ((semg0, semg1), (semo0, semo1)), acc_s)
  plsc.subcore_barrier()
  _flush_acc(buf, acc_s, out_o, base, cid, sid)


_SC_OUT = [jax.ShapeDtypeStruct((_NC, _N, 128), jnp.float32)]


def _pipe_scratch(nidx):
  return [
      pltpu.VMEM((nidx * _EW,), jnp.int32),
      pltpu.VMEM((_EW,), jnp.int32),
      pltpu.VMEM((_EW,), jnp.int32),
      pltpu.VMEM((_EW, 128), jnp.float32),
      pltpu.VMEM((_EW, 128), jnp.float32),
      pltpu.VMEM((24, 128), jnp.float32),
      pltpu.SemaphoreType.DMA,
      pltpu.SemaphoreType.DMA,
      pltpu.SemaphoreType.DMA,
      pltpu.SemaphoreType.DMA,
  ]


_SC_SCRATCH_IDX1 = [
    pltpu.VMEM((_EW,), jnp.int32),
    pltpu.VMEM((_EW,), jnp.int32),
    pltpu.VMEM((_EW, 128), jnp.float32),
    pltpu.VMEM((24, 128), jnp.float32),
    pltpu.SemaphoreType.DMA,
    pltpu.SemaphoreType.DMA,
]


@functools.cache
def _sc_kernels():
  shared = [pltpu.VMEM_SHARED((_N, 128), jnp.float32)]
  ones = pl.kernel(_sc_ones_body, out_type=_SC_OUT, mesh=_mesh(),
                   scratch_types=_SC_SCRATCH_IDX1 + shared)
  agg_split = pl.kernel(_sc_agg_split_body, out_type=_SC_OUT, mesh=_mesh(),
                        scratch_types=_pipe_scratch(_PT_HALF) + shared)
  agg_cols = pl.kernel(_sc_agg_cols_body, out_type=_SC_OUT, mesh=_mesh(),
                       scratch_types=_pipe_scratch(_PT_ALL) + shared)
  s_pass = pl.kernel(_sc_s_body, out_type=_SC_OUT, mesh=_mesh(),
                     scratch_types=_pipe_scratch(_PT_HALF) + shared)
  return ones, agg_split, agg_cols, s_pass


_R = 400
_G = _N // _R


def _elu(y):
  return jnp.where(y > 0.0, y, jnp.exp(jnp.minimum(y, 0.0)) - 1.0)


def _layer_norm(y, g, b):
  m = jnp.mean(y, axis=-1, keepdims=True)
  v = jnp.mean((y - m) * (y - m), axis=-1, keepdims=True)
  return (y - m) / jnp.sqrt(v + 1e-5) * g + b


def _t0_body(cnt, inv_o):
  inv_o[...] = 1.0 / (cnt[0] + cnt[1] + 1e-8)


def _t1_body(x, agg, inv128, w1t, b1, g1, be1, h1l_o, h1r_o):
  inv = inv128[:, 0:1]
  a = (agg[0] + agg[1]) * inv
  y = jnp.dot(x[...] + a, w1t[...], preferred_element_type=jnp.float32) + b1[...]
  h = _elu(_layer_norm(y, g1[...], be1[...]))
  h1l_o[...] = h[:, :128]
  h1r_o[...] = h[:, 128:]


def _t2_body(h1l, h1r, agg, inv128, s128, w2t, b2, g2, be2, tp_o):
  inv = inv128[:, 0:1]
  h1 = jnp.concatenate([h1l[...], h1r[...]], axis=1)
  a = jnp.concatenate([agg[0], agg[1]], axis=1) * inv
  y = jnp.dot(h1 + a, w2t[...], preferred_element_type=jnp.float32) + b2[...]
  h2 = _elu(_layer_norm(y, g2[...], be2[...]))
  w = (1.0 + s128[0, :, 0] + s128[1, :, 0]) * (1.0 / _N)
  tp_o[...] = jnp.sum(h2 * w[:, None], axis=0)[None, None, :]


def _t3_body(tp, w3t, b3, wm1at, wm1bt, bm1, wm2t, bm2, out_o):
  t = jnp.sum(tp[...][:, 0, :], axis=0, keepdims=True)
  g = jnp.dot(t, w3t[...], preferred_element_type=jnp.float32) + b3[...]
  z = jnp.maximum(
      jnp.dot(g, wm1at[...], preferred_element_type=jnp.float32)
      + jnp.dot(g, wm1bt[...], preferred_element_type=jnp.float32)
      + bm1[...], 0.0)
  out_o[...] = jnp.dot(z, wm2t[...], preferred_element_type=jnp.float32) + bm2[...]


_tc0 = pl.pallas_call(
    _t0_body,
    grid=(_G,),
    in_specs=[pl.BlockSpec((_NC, _R, 128), lambda i: (0, i, 0))],
    out_specs=pl.BlockSpec((_R, 128), lambda i: (i, 0)),
    out_shape=jax.ShapeDtypeStruct((_N, 128), jnp.float32),
)

_tc1_in_specs = [
    pl.BlockSpec((_R, 128), lambda i: (i, 0)),
    pl.BlockSpec((_NC, _R, 128), lambda i: (0, i, 0)),
    pl.BlockSpec((_R, 128), lambda i: (i, 0)),
    pl.BlockSpec((128, 256), lambda i: (0, 0)),
    pl.BlockSpec((1, 256), lambda i: (0, 0)),
    pl.BlockSpec((1, 256), lambda i: (0, 0)),
    pl.BlockSpec((1, 256), lambda i: (0, 0)),
]
_tc1_out_specs = [
    pl.BlockSpec((_R, 128), lambda i: (i, 0)),
    pl.BlockSpec((_R, 128), lambda i: (i, 0)),
]

_tc1 = pl.pallas_call(
    _t1_body,
    grid=(_G,),
    in_specs=_tc1_in_specs,
    out_specs=_tc1_out_specs,
    out_shape=[jax.ShapeDtypeStruct((_N, 128), jnp.float32)] * 2,
)

_tc2_in_specs = [
    pl.BlockSpec((_R, 128), lambda i: (i, 0)),
    pl.BlockSpec((_R, 128), lambda i: (i, 0)),
    pl.BlockSpec((_NC, _R, 128), lambda i: (0, i, 0)),
    pl.BlockSpec((_R, 128), lambda i: (i, 0)),
    pl.BlockSpec((_NC, _R, 128), lambda i: (0, i, 0)),
    pl.BlockSpec((256, 256), lambda i: (0, 0)),
    pl.BlockSpec((1, 256), lambda i: (0, 0)),
    pl.BlockSpec((1, 256), lambda i: (0, 0)),
    pl.BlockSpec((1, 256), lambda i: (0, 0)),
]

_tc2 = pl.pallas_call(
    _t2_body,
    grid=(_G,),
    in_specs=_tc2_in_specs,
    out_specs=pl.BlockSpec((1, 1, 256), lambda i: (i, 0, 0)),
    out_shape=jax.ShapeDtypeStruct((_G, 1, 256), jnp.float32),
)

_tc3_in_specs = [
    pl.BlockSpec((_G, 1, 256), lambda i: (0, 0, 0)),
    pl.BlockSpec((256, 256), lambda i: (0, 0)),
    pl.BlockSpec((1, 256), lambda i: (0, 0)),
    pl.BlockSpec((256, 256), lambda i: (0, 0)),
    pl.BlockSpec((256, 256), lambda i: (0, 0)),
    pl.BlockSpec((1, 256), lambda i: (0, 0)),
    pl.BlockSpec((256, 256), lambda i: (0, 0)),
    pl.BlockSpec((1, 256), lambda i: (0, 0)),
]

_tc3 = pl.pallas_call(
    _t3_body,
    grid=(1,),
    in_specs=_tc3_in_specs,
    out_specs=pl.BlockSpec((1, 256), lambda i: (0, 0)),
    out_shape=jax.ShapeDtypeStruct((1, 256), jnp.float32),
)


@jax.jit
def kernel(x, edge_index, W1, b1, ln1_g, ln1_b, W2, b2, ln2_g, ln2_b,
           W3, b3, Wm1, bm1, Wm2, bm2):
  src = edge_index[0]
  dst = edge_index[1]
  ones_k, agg_split_k, agg_cols_k, s_k = _sc_kernels()

  (cnt2,) = ones_k(dst)
  inv128 = _tc0(cnt2)
  (agg0,) = agg_split_k(src, dst, x, cnt2)
  (s2,) = s_k(src, dst, inv128, agg0)
  h1l, h1r = _tc1(x, agg0, inv128, W1.T, b1[None], ln1_g[None], ln1_b[None])
  (agg1,) = agg_cols_k(src, dst, h1l, h1r, s2)
  tp = _tc2(h1l, h1r, agg1, inv128, s2, W2.T, b2[None], ln2_g[None],
            ln2_b[None])
  out = _tc3(tp, W3.T, b3[None], Wm1[:, :256].T, Wm1[:, 256:].T, bm1[None],
             Wm2.T, bm2[None])
  return out
